# baseline (speedup 1.0000x reference)
import functools

import jax
import jax.numpy as jnp
from jax import lax
from jax.experimental import pallas as pl
from jax.experimental.pallas import tpu as pltpu
from jax.experimental.pallas import tpu_sc as plsc

TYPES_NUM = 100000
DIM = 128

NW = 32
CHUNK = 400
NCHUNK = TYPES_NUM // CHUNK


@functools.partial(
    pl.kernel,
    mesh=plsc.VectorSubcoreMesh(core_axis_name="c", subcore_axis_name="s"),
    out_type=jax.ShapeDtypeStruct((TYPES_NUM, DIM), jnp.float32),
    scratch_types=[pltpu.VMEM((CHUNK, DIM), jnp.float32)],
)
def _sc_centers(bw_hbm, out_hbm, buf):
    wid = lax.axis_index("s") * 2 + lax.axis_index("c")
    nmine = (NCHUNK - wid + NW - 1) // NW

    def body(k, carry):
        r0 = (wid + k * NW) * CHUNK
        pltpu.sync_copy(bw_hbm.at[pl.ds(r0, CHUNK), pl.ds(0, DIM)], buf)
        pltpu.sync_copy(buf, out_hbm.at[pl.ds(r0, CHUNK)])
        return carry

    lax.fori_loop(0, nmine, body, 0)


ROWS = 1000


def _off_body(x_ref, o_ref):
    o_ref[...] = jnp.maximum(x_ref[...], 0.0) + 1e-6


def kernel(box_weight):
    n = box_weight.shape[0]
    centers = _sc_centers(box_weight)
    offsets = pl.pallas_call(
        _off_body,
        grid=(n // ROWS,),
        in_specs=[pl.BlockSpec((ROWS, DIM), lambda i: (i, 1))],
        out_specs=pl.BlockSpec((ROWS, DIM), lambda i: (i, 0)),
        out_shape=jax.ShapeDtypeStruct((n, DIM), jnp.float32),
    )(box_weight)
    return (centers, offsets)
